# baseline (speedup 1.0000x reference)
import jax
import jax.numpy as jnp
from jax.experimental import pallas as pl
from jax.experimental.pallas import tpu as pltpu

N_BASES = 4
L = 1024
BI = 512


def _body(tab_ref, seqc_ref, seqr_ref, out_ref):
    c = pl.program_id(0)
    cm = jax.lax.rem(c, N_BASES)
    seqc = seqc_ref[...]
    seqr = seqr_ref[...]
    acc_i = jnp.zeros((BI, 1), jnp.float32)
    acc_j = jnp.zeros((1, L), jnp.float32)
    for k in range(N_BASES):
        t = tab_ref[k, cm]
        acc_i += t * (seqc == k).astype(jnp.float32)
        acc_j += t * (seqr == k).astype(jnp.float32)
    @pl.when(c < N_BASES)
    def _():
        out_ref[0] = jnp.broadcast_to(acc_i, (BI, L))

    @pl.when(c >= N_BASES)
    def _():
        out_ref[0] = jnp.broadcast_to(acc_j, (BI, L))


def kernel(sequence, base_table):
    seq_col = sequence.reshape(L, 1)
    seq_row = sequence.reshape(1, L)
    out = pl.pallas_call(
        _body,
        grid=(2 * N_BASES, L // BI),
        in_specs=[
            pl.BlockSpec(memory_space=pltpu.SMEM),
            pl.BlockSpec((BI, 1), lambda c, i: (i, 0)),
            pl.BlockSpec((1, L), lambda c, i: (0, 0)),
        ],
        out_specs=pl.BlockSpec((1, BI, L), lambda c, i: (c, i, 0)),
        out_shape=jax.ShapeDtypeStruct((2 * N_BASES, L, L), jnp.float32),
    )(base_table, seq_col, seq_row)
    return out[None]


# speedup vs baseline: 1.4016x; 1.4016x over previous
import jax
import jax.numpy as jnp
from jax.experimental import pallas as pl
from jax.experimental.pallas import tpu as pltpu

N_BASES = 4
L = 1024
SUB = 128
NREP = L // SUB


def _body(tab_ref, seqc_ref, seqr_ref, out_ref, ibuf, jbuf, isem, jsem):
    seqc = seqc_ref[...]
    seqr = seqr_ref[...]
    for c in range(N_BASES):
        acc_j = jnp.zeros((1, L), jnp.float32)
        for k in range(N_BASES):
            acc_j += tab_ref[k, c] * (seqr == k).astype(jnp.float32)
        jbuf[c] = jnp.broadcast_to(acc_j, (SUB, L))
    for c in range(N_BASES):
        for r in range(NREP):
            pltpu.make_async_copy(
                jbuf.at[c],
                out_ref.at[N_BASES + c, pl.ds(r * SUB, SUB), :],
                jsem.at[c, r],
            ).start()
    for c in range(N_BASES):
        acc_i = jnp.zeros((L, 1), jnp.float32)
        for k in range(N_BASES):
            acc_i += tab_ref[k, c] * (seqc == k).astype(jnp.float32)
        ibuf[c] = jnp.broadcast_to(acc_i, (L, L))
        pltpu.make_async_copy(ibuf.at[c], out_ref.at[c], isem.at[c]).start()
    for c in range(N_BASES):
        for r in range(NREP):
            pltpu.make_async_copy(
                jbuf.at[c],
                out_ref.at[N_BASES + c, pl.ds(r * SUB, SUB), :],
                jsem.at[c, r],
            ).wait()
    for c in range(N_BASES):
        pltpu.make_async_copy(ibuf.at[c], out_ref.at[c], isem.at[c]).wait()


def kernel(sequence, base_table):
    seq_col = sequence.reshape(L, 1)
    seq_row = sequence.reshape(1, L)
    out = pl.pallas_call(
        _body,
        in_specs=[
            pl.BlockSpec(memory_space=pltpu.SMEM),
            pl.BlockSpec(memory_space=pltpu.VMEM),
            pl.BlockSpec(memory_space=pltpu.VMEM),
        ],
        out_specs=pl.BlockSpec(memory_space=pl.ANY),
        out_shape=jax.ShapeDtypeStruct((2 * N_BASES, L, L), jnp.float32),
        scratch_shapes=[
            pltpu.VMEM((N_BASES, L, L), jnp.float32),
            pltpu.VMEM((N_BASES, SUB, L), jnp.float32),
            pltpu.SemaphoreType.DMA((N_BASES,)),
            pltpu.SemaphoreType.DMA((N_BASES, NREP)),
        ],
    )(base_table, seq_col, seq_row)
    return out[None]
